# Initial kernel scaffold; baseline (speedup 1.0000x reference)
import jax
import jax.numpy as jnp
from jax.experimental import pallas as pl


def kernel(points, W_d, b_d, W_a, b_a):
    raise NotImplementedError("write your pallas kernel here")


# speedup vs baseline: 24.7026x; 24.7026x over previous
import numpy as np
import jax
import jax.numpy as jnp
from jax.experimental import pallas as pl

_H = 256
_N = 256
_K = 3
_SIGMA_D = 0.2
_FACTOR_A = 180.0 / (15.0 * np.pi)
_TI = 8
_NT = _N // _TI

_DIV_TERM = np.exp(
    np.arange(0, _H, 2, dtype=np.float32) * np.float32(-np.log(10000.0) / _H)
).astype(np.float32)


def _knn_kernel(pts_ref, dist_ref, oh_ref):
    pts = pts_ref[...]
    xy = jax.lax.dot_general(pts, pts, (((1,), (1,)), ((), ())),
                             preferred_element_type=jnp.float32)
    x2c = jnp.sum(pts * pts, axis=1, keepdims=True)
    y2r = jnp.sum(pts * pts, axis=1)[None, :]
    sq = jnp.maximum(x2c - 2.0 * xy + y2r, 0.0)
    dist = jnp.sqrt(sq)
    dist_ref[...] = dist
    neg = -dist
    jota = jax.lax.broadcasted_iota(jnp.int32, (_N, _N), 1)
    for kk in range(_K + 1):
        m = jnp.max(neg, axis=1, keepdims=True)
        cand = jnp.where(neg == m, jota, _N)
        sel = jnp.min(cand, axis=1, keepdims=True)
        if kk > 0:
            oh_ref[kk - 1] = (jota == sel).astype(jnp.float32)
        neg = jnp.where(jota == sel, -jnp.inf, neg)


def _fused(pts_ref, ptst_ref, distt_ref, oht_ref, wd_ref, wa_ref,
           bd_ref, ba_ref, div_ref, out_ref):
    pts = pts_ref[...]
    pit = ptst_ref[0]
    distt = distt_ref[0]
    d_idxt = distt / _SIGMA_D

    crow = [pts[:, c:c + 1] for c in range(3)]
    pirow = [pit[c:c + 1, :] for c in range(3)]
    anc = [crow[c] - pirow[c] for c in range(3)]

    div = div_ref[...]
    bd = bd_ref[...]
    ba = ba_ref[...]

    a_idxts = []
    for kk in range(_K):
        mask = oht_ref[0, kk]
        r = [jnp.sum(mask * crow[c], axis=0, keepdims=True)
             - pirow[c] for c in range(3)]
        c1 = r[1] * anc[2] - r[2] * anc[1]
        c2 = r[2] * anc[0] - r[0] * anc[2]
        c3 = r[0] * anc[1] - r[1] * anc[0]
        sinv = jnp.sqrt(c1 * c1 + c2 * c2 + c3 * c3)
        cosv = r[0] * anc[0] + r[1] * anc[1] + r[2] * anc[2]
        a_idxts.append(jnp.arctan2(sinv, cosv) * _FACTOR_A)

    def embed_mm(col, w_ref):
        om = col * div
        feats = jnp.concatenate([jnp.sin(om), jnp.cos(om)], axis=1)
        return jax.lax.dot_general(feats, w_ref[...],
                                   (((1,), (1,)), ((), ())),
                                   preferred_element_type=jnp.float32)

    for ii in range(_TI):
        e_d = embed_mm(d_idxt[:, ii:ii + 1], wd_ref)
        amax = None
        for kk in range(_K):
            e_a = embed_mm(a_idxts[kk][:, ii:ii + 1], wa_ref)
            amax = e_a if amax is None else jnp.maximum(amax, e_a)
        out_ref[0, ii] = (e_d + bd) + (amax + ba)


def kernel(points, W_d, b_d, W_a, b_a):
    pts = jnp.zeros((_N, 8), jnp.float32).at[:, :3].set(points[0])

    dist, oh = pl.pallas_call(
        _knn_kernel,
        out_shape=(jax.ShapeDtypeStruct((_N, _N), jnp.float32),
                   jax.ShapeDtypeStruct((_K, _N, _N), jnp.float32)),
    )(pts)

    distt = dist.T.reshape(_N, _NT, _TI).transpose(1, 0, 2)
    oht = (jnp.swapaxes(oh, 1, 2)
           .reshape(_K, _N, _NT, _TI).transpose(2, 0, 1, 3))
    ptst = pts.reshape(_NT, _TI, 8).transpose(0, 2, 1)

    wd = jnp.concatenate([W_d[:, 0::2], W_d[:, 1::2]], axis=1)
    wa = jnp.concatenate([W_a[:, 0::2], W_a[:, 1::2]], axis=1)
    bd = b_d.reshape(1, _H)
    ba = b_a.reshape(1, _H)
    div = jnp.asarray(_DIV_TERM).reshape(1, _H // 2)

    return pl.pallas_call(
        _fused,
        grid=(_NT,),
        in_specs=[
            pl.BlockSpec((_N, 8), lambda i: (0, 0)),
            pl.BlockSpec((1, 8, _TI), lambda i: (i, 0, 0)),
            pl.BlockSpec((1, _N, _TI), lambda i: (i, 0, 0)),
            pl.BlockSpec((1, _K, _N, _TI), lambda i: (i, 0, 0, 0)),
            pl.BlockSpec((_H, _H), lambda i: (0, 0)),
            pl.BlockSpec((_H, _H), lambda i: (0, 0)),
            pl.BlockSpec((1, _H), lambda i: (0, 0)),
            pl.BlockSpec((1, _H), lambda i: (0, 0)),
            pl.BlockSpec((1, _H // 2), lambda i: (0, 0)),
        ],
        out_specs=pl.BlockSpec((1, _TI, _N, _H), lambda i: (0, i, 0, 0)),
        out_shape=jax.ShapeDtypeStruct((1, _N, _N, _H), jnp.float32),
    )(pts, ptst, distt, oht, wd, wa, bd, ba, div)


# speedup vs baseline: 24.7276x; 1.0010x over previous
import numpy as np
import jax
import jax.numpy as jnp
from jax.experimental import pallas as pl
from jax.experimental.pallas import tpu as pltpu

_H = 256
_N = 256
_K = 3
_SIGMA_D = 0.2
_FACTOR_A = 180.0 / (15.0 * np.pi)
_TI = 8
_NT = _N // _TI

_DIV_TERM = np.exp(
    np.arange(0, _H, 2, dtype=np.float32) * np.float32(-np.log(10000.0) / _H)
).astype(np.float32)


def _knn_kernel(pts_ref, dist_ref, oh_ref):
    pts = pts_ref[...]
    xy = jax.lax.dot_general(pts, pts, (((1,), (1,)), ((), ())),
                             preferred_element_type=jnp.float32)
    x2c = jnp.sum(pts * pts, axis=1, keepdims=True)
    y2r = jnp.sum(pts * pts, axis=1)[None, :]
    sq = jnp.maximum(x2c - 2.0 * xy + y2r, 0.0)
    dist = jnp.sqrt(sq)
    dist_ref[...] = dist
    neg = -dist
    jota = jax.lax.broadcasted_iota(jnp.int32, (_N, _N), 1)
    for kk in range(_K + 1):
        m = jnp.max(neg, axis=1, keepdims=True)
        cand = jnp.where(neg == m, jota, _N)
        sel = jnp.min(cand, axis=1, keepdims=True)
        if kk > 0:
            oh_ref[kk - 1] = (jota == sel).astype(jnp.float32)
        neg = jnp.where(jota == sel, -jnp.inf, neg)


def _fused(pts_ref, ptst_ref, distt_ref, oht_ref, wd_ref, wa_ref,
           bd_ref, ba_ref, div_ref, out_ref):
    pts = pts_ref[...]
    pit = ptst_ref[0]
    distt = distt_ref[0]
    d_idxt = distt / _SIGMA_D

    crow = [pts[:, c:c + 1] for c in range(3)]
    pirow = [pit[c:c + 1, :] for c in range(3)]
    anc = [crow[c] - pirow[c] for c in range(3)]

    div = div_ref[...]
    bd = bd_ref[...]
    ba = ba_ref[...]

    a_idxts = []
    for kk in range(_K):
        mask = oht_ref[0, kk]
        r = [jnp.sum(mask * crow[c], axis=0, keepdims=True)
             - pirow[c] for c in range(3)]
        c1 = r[1] * anc[2] - r[2] * anc[1]
        c2 = r[2] * anc[0] - r[0] * anc[2]
        c3 = r[0] * anc[1] - r[1] * anc[0]
        sinv = jnp.sqrt(c1 * c1 + c2 * c2 + c3 * c3)
        cosv = r[0] * anc[0] + r[1] * anc[1] + r[2] * anc[2]
        a_idxts.append(jnp.arctan2(sinv, cosv) * _FACTOR_A)

    def feats_of(cols):
        om = jnp.concatenate([c * div for c in cols], axis=0)
        return jnp.concatenate([jnp.sin(om), jnp.cos(om)], axis=1)

    def mm(feats, w_ref):
        return jax.lax.dot_general(feats, w_ref[...],
                                   (((1,), (1,)), ((), ())),
                                   preferred_element_type=jnp.float32)

    m = _TI * _N
    e_d = mm(feats_of([d_idxt[:, ii:ii + 1] for ii in range(_TI)]), wd_ref)
    e_a = mm(feats_of([a_idxts[kk][:, ii:ii + 1]
                       for kk in range(_K) for ii in range(_TI)]), wa_ref)
    amax = jnp.maximum(jnp.maximum(e_a[:m], e_a[m:2 * m]), e_a[2 * m:])
    total = (e_d + bd) + (amax + ba)
    out_ref[...] = total.reshape(1, _TI, _N, _H)


def kernel(points, W_d, b_d, W_a, b_a):
    pts = jnp.zeros((_N, 8), jnp.float32).at[:, :3].set(points[0])

    dist, oh = pl.pallas_call(
        _knn_kernel,
        out_shape=(jax.ShapeDtypeStruct((_N, _N), jnp.float32),
                   jax.ShapeDtypeStruct((_K, _N, _N), jnp.float32)),
    )(pts)

    distt = dist.T.reshape(_N, _NT, _TI).transpose(1, 0, 2)
    oht = (jnp.swapaxes(oh, 1, 2)
           .reshape(_K, _N, _NT, _TI).transpose(2, 0, 1, 3))
    ptst = pts.reshape(_NT, _TI, 8).transpose(0, 2, 1)

    wd = jnp.concatenate([W_d[:, 0::2], W_d[:, 1::2]], axis=1)
    wa = jnp.concatenate([W_a[:, 0::2], W_a[:, 1::2]], axis=1)
    bd = b_d.reshape(1, _H)
    ba = b_a.reshape(1, _H)
    div = jnp.asarray(_DIV_TERM).reshape(1, _H // 2)

    return pl.pallas_call(
        _fused,
        grid=(_NT,),
        in_specs=[
            pl.BlockSpec((_N, 8), lambda i: (0, 0)),
            pl.BlockSpec((1, 8, _TI), lambda i: (i, 0, 0)),
            pl.BlockSpec((1, _N, _TI), lambda i: (i, 0, 0)),
            pl.BlockSpec((1, _K, _N, _TI), lambda i: (i, 0, 0, 0)),
            pl.BlockSpec((_H, _H), lambda i: (0, 0)),
            pl.BlockSpec((_H, _H), lambda i: (0, 0)),
            pl.BlockSpec((1, _H), lambda i: (0, 0)),
            pl.BlockSpec((1, _H), lambda i: (0, 0)),
            pl.BlockSpec((1, _H // 2), lambda i: (0, 0)),
        ],
        out_specs=pl.BlockSpec((1, _TI, _N, _H), lambda i: (0, i, 0, 0)),
        out_shape=jax.ShapeDtypeStruct((1, _N, _N, _H), jnp.float32),
        compiler_params=pltpu.CompilerParams(
            dimension_semantics=("parallel",)),
    )(pts, ptst, distt, oht, wd, wa, bd, ba, div)


# speedup vs baseline: 53.7679x; 2.1744x over previous
import numpy as np
import jax
import jax.numpy as jnp
from jax.experimental import pallas as pl
from jax.experimental.pallas import tpu as pltpu

_H = 256
_N = 256
_K = 3
_SIGMA_D = 0.2
_FACTOR_A = 180.0 / (15.0 * np.pi)
_TI = 8
_NT = _N // _TI

_DIV_TERM = np.exp(
    np.arange(0, _H, 2, dtype=np.float32) * np.float32(-np.log(10000.0) / _H)
).astype(np.float32)

_INV_PIO2 = np.float32(2.0 / np.pi)
_PIO2_HI = np.float32(1.5707963267948966)
_PIO2_LO = np.float32(1.5707963267948966 - float(np.float32(1.5707963267948966)))
_S1, _S2, _S3 = (np.float32(-1.6666654611e-1), np.float32(8.3321608736e-3),
                 np.float32(-1.9515295891e-4))
_C1, _C2, _C3 = (np.float32(4.166664568298827e-2),
                 np.float32(-1.388731625493765e-3),
                 np.float32(2.443315711809948e-5))


def _sincos(u):
    n = jnp.round(u * _INV_PIO2)
    q = n.astype(jnp.int32)
    r = (u - n * _PIO2_HI) - n * _PIO2_LO
    r2 = r * r
    sin_r = r + r * r2 * (_S1 + r2 * (_S2 + r2 * _S3))
    cos_r = 1.0 + r2 * (-0.5 + r2 * (_C1 + r2 * (_C2 + r2 * _C3)))
    swap = jax.lax.bitwise_and(q, 1) == 1
    s = jnp.where(swap, cos_r, sin_r)
    c = jnp.where(swap, sin_r, cos_r)
    s = jnp.where(jax.lax.bitwise_and(q, 2) == 2, -s, s)
    c = jnp.where(jax.lax.bitwise_and(q + 1, 2) == 2, -c, c)
    return s, c


def _knn_kernel(pts_ref, dist_ref, oh_ref):
    pts = pts_ref[...]
    xy = jax.lax.dot_general(pts, pts, (((1,), (1,)), ((), ())),
                             preferred_element_type=jnp.float32)
    x2c = jnp.sum(pts * pts, axis=1, keepdims=True)
    y2r = jnp.sum(pts * pts, axis=1)[None, :]
    sq = jnp.maximum(x2c - 2.0 * xy + y2r, 0.0)
    dist = jnp.sqrt(sq)
    dist_ref[...] = dist
    neg = -dist
    jota = jax.lax.broadcasted_iota(jnp.int32, (_N, _N), 1)
    for kk in range(_K + 1):
        m = jnp.max(neg, axis=1, keepdims=True)
        cand = jnp.where(neg == m, jota, _N)
        sel = jnp.min(cand, axis=1, keepdims=True)
        if kk > 0:
            oh_ref[kk - 1] = (jota == sel).astype(jnp.float32)
        neg = jnp.where(jota == sel, -jnp.inf, neg)


def _fused(pts_ref, ptst_ref, distt_ref, oht_ref, wd_ref, wa_ref,
           bd_ref, ba_ref, div_ref, out_ref):
    pts = pts_ref[...]
    pit = ptst_ref[0]
    distt = distt_ref[0]
    d_idxt = distt / _SIGMA_D

    crow = [pts[:, c:c + 1] for c in range(3)]
    pirow = [pit[c:c + 1, :] for c in range(3)]
    anc = [crow[c] - pirow[c] for c in range(3)]

    div = div_ref[...]
    bd = bd_ref[...]
    ba = ba_ref[...]

    a_idxts = []
    for kk in range(_K):
        mask = oht_ref[0, kk]
        r = [jnp.sum(mask * crow[c], axis=0, keepdims=True)
             - pirow[c] for c in range(3)]
        c1 = r[1] * anc[2] - r[2] * anc[1]
        c2 = r[2] * anc[0] - r[0] * anc[2]
        c3 = r[0] * anc[1] - r[1] * anc[0]
        sinv = jnp.sqrt(c1 * c1 + c2 * c2 + c3 * c3)
        cosv = r[0] * anc[0] + r[1] * anc[1] + r[2] * anc[2]
        a_idxts.append(jnp.arctan2(sinv, cosv) * _FACTOR_A)

    def feats_of(cols):
        om = jnp.concatenate([c * div for c in cols], axis=0)
        s, c = _sincos(om)
        return jnp.concatenate([s, c], axis=1)

    def mm(feats, w_ref):
        return jax.lax.dot_general(feats, w_ref[...],
                                   (((1,), (1,)), ((), ())),
                                   preferred_element_type=jnp.float32)

    m = _TI * _N
    e_d = mm(feats_of([d_idxt[:, ii:ii + 1] for ii in range(_TI)]), wd_ref)
    e_a = mm(feats_of([a_idxts[kk][:, ii:ii + 1]
                       for kk in range(_K) for ii in range(_TI)]), wa_ref)
    amax = jnp.maximum(jnp.maximum(e_a[:m], e_a[m:2 * m]), e_a[2 * m:])
    total = (e_d + bd) + (amax + ba)
    out_ref[...] = total.reshape(1, _TI, _N, _H)


def kernel(points, W_d, b_d, W_a, b_a):
    pts = jnp.zeros((_N, 8), jnp.float32).at[:, :3].set(points[0])

    dist, oh = pl.pallas_call(
        _knn_kernel,
        out_shape=(jax.ShapeDtypeStruct((_N, _N), jnp.float32),
                   jax.ShapeDtypeStruct((_K, _N, _N), jnp.float32)),
    )(pts)

    distt = dist.T.reshape(_N, _NT, _TI).transpose(1, 0, 2)
    oht = (jnp.swapaxes(oh, 1, 2)
           .reshape(_K, _N, _NT, _TI).transpose(2, 0, 1, 3))
    ptst = pts.reshape(_NT, _TI, 8).transpose(0, 2, 1)

    wd = jnp.concatenate([W_d[:, 0::2], W_d[:, 1::2]], axis=1)
    wa = jnp.concatenate([W_a[:, 0::2], W_a[:, 1::2]], axis=1)
    bd = b_d.reshape(1, _H)
    ba = b_a.reshape(1, _H)
    div = jnp.asarray(_DIV_TERM).reshape(1, _H // 2)

    return pl.pallas_call(
        _fused,
        grid=(_NT,),
        in_specs=[
            pl.BlockSpec((_N, 8), lambda i: (0, 0)),
            pl.BlockSpec((1, 8, _TI), lambda i: (i, 0, 0)),
            pl.BlockSpec((1, _N, _TI), lambda i: (i, 0, 0)),
            pl.BlockSpec((1, _K, _N, _TI), lambda i: (i, 0, 0, 0)),
            pl.BlockSpec((_H, _H), lambda i: (0, 0)),
            pl.BlockSpec((_H, _H), lambda i: (0, 0)),
            pl.BlockSpec((1, _H), lambda i: (0, 0)),
            pl.BlockSpec((1, _H), lambda i: (0, 0)),
            pl.BlockSpec((1, _H // 2), lambda i: (0, 0)),
        ],
        out_specs=pl.BlockSpec((1, _TI, _N, _H), lambda i: (0, i, 0, 0)),
        out_shape=jax.ShapeDtypeStruct((1, _N, _N, _H), jnp.float32),
        compiler_params=pltpu.CompilerParams(
            dimension_semantics=("parallel",)),
    )(pts, ptst, distt, oht, wd, wa, bd, ba, div)
